# baseline (speedup 1.0000x reference)
import functools

import jax
import jax.numpy as jnp
from jax import lax
from jax.experimental import pallas as pl
from jax.experimental.pallas import tpu as pltpu
from jax.experimental.pallas import tpu_sc as plsc

N = 100000
V = 20000
D_IN = 4
H = 256
CS = 96
NCLS = 19

N_PAD = 102400
V_PAD = 20480
BN = 256
BV = 128

NC, NS = 2, 16
T1 = N_PAD // NS
C1 = T1 // 128
T3 = N_PAD // (NC * NS)
C3 = T3 // 128
VROWS = V_PAD // NS

_mesh = plsc.VectorSubcoreMesh(core_axis_name="c", subcore_axis_name="s")
_sc_params = pltpu.CompilerParams(use_tc_tiling_on_sc=False,
                                  needs_layout_passes=False)


def _s0_body(x_ref, w0_ref, b0_ref, h_ref):
    i = pl.program_id(0)
    h = jnp.dot(x_ref[...], w0_ref[...], preferred_element_type=jnp.float32)
    h = jnp.maximum(h + b0_ref[...], 0.0)
    rows = i * BN + lax.broadcasted_iota(jnp.int32, (BN, 1), 0)
    h_ref[...] = jnp.where(rows < N, h, 0.0)


def _s0(x_pad, w0, b0r):
    return pl.pallas_call(
        _s0_body,
        grid=(N_PAD // BN,),
        in_specs=[
            pl.BlockSpec((BN, D_IN), lambda i: (i, 0)),
            pl.BlockSpec((D_IN, H), lambda i: (0, 0)),
            pl.BlockSpec((1, H), lambda i: (0, 0)),
        ],
        out_specs=pl.BlockSpec((BN, H), lambda i: (i, 0)),
        out_shape=jax.ShapeDtypeStruct((N_PAD, H), jnp.float32),
    )(x_pad, w0, b0r)


def _s1_body(h_hbm, idx_hbm, vsum_hbm, cntp_hbm,
             acc, hbuf, idxbuf, zbuf, cntbuf):
    c = lax.axis_index("c")
    s = lax.axis_index("s")
    zero16 = jnp.zeros((16,), jnp.float32)

    def _zrow(i, _):
        for k in range(4):
            zbuf[i, pl.ds(k * 16, 16)] = zero16
        return 0
    lax.fori_loop(0, 128, _zrow, 0)

    def _zcnt(i, _):
        cntbuf[pl.ds(i * 16, 16)] = zero16
        return 0
    lax.fori_loop(0, V_PAD // 16, _zcnt, 0)

    pltpu.sync_copy(idx_hbm.at[pl.ds(s * C1, C1)], idxbuf)

    for p in range(2):
        colbase = c * (2 * 64) + p * 64
        def _zacc(i, _):
            pltpu.sync_copy(zbuf, acc.at[pl.ds(s * VROWS + i * 128, 128), :])
            return 0
        lax.fori_loop(0, VROWS // 128, _zacc, 0)
        plsc.subcore_barrier()

        def _chunk(j, _):
            rowstart = s * T1 + j * 128
            pltpu.sync_copy(
                h_hbm.at[pl.ds(rowstart, 128), pl.ds(colbase, 64)], hbuf)
            pltpu.sync_copy(hbuf, acc.at[idxbuf.at[j]], add=True)
            return 0
        lax.fori_loop(0, C1, _chunk, 0)
        plsc.subcore_barrier()

        pltpu.sync_copy(
            acc.at[pl.ds(s * VROWS, VROWS), :],
            vsum_hbm.at[pl.ds(s * VROWS, VROWS), pl.ds(colbase, 64)])

    @pl.when(c == 0)
    def _():
        ones = jnp.ones((16,), jnp.float32)
        iota = lax.iota(jnp.int32, 16)

        def _cchunk(j, _):
            def _cvec(k, _):
                gbase = s * T1 + j * 128 + k * 16
                iv = idxbuf[j, pl.ds(k * 16, 16)]
                mask = (gbase + iota) < N
                plsc.addupdate_scatter(cntbuf, [iv], ones, mask=mask)
                return 0
            lax.fori_loop(0, 8, _cvec, 0)
            return 0
        lax.fori_loop(0, C1, _cchunk, 0)
        pltpu.sync_copy(cntbuf, cntp_hbm.at[s])


def _s1(h, idx2d):
    return pl.kernel(
        _s1_body,
        out_type=[
            jax.ShapeDtypeStruct((V_PAD, H), jnp.float32),
            jax.ShapeDtypeStruct((NS, V_PAD), jnp.float32),
        ],
        mesh=_mesh,
        scratch_types=[
            pltpu.VMEM_SHARED((V_PAD, 64), jnp.float32),
            pltpu.VMEM((128, 64), jnp.float32),
            pltpu.VMEM((C1, 128), jnp.int32),
            pltpu.VMEM((128, 64), jnp.float32),
            pltpu.VMEM((V_PAD,), jnp.float32),
        ],
        compiler_params=_sc_params,
    )(h, idx2d)


def _s2_body(vsum_ref, cntp_ref, wv_ref, bv_ref, w1_ref, voxp_ref):
    cnt = jnp.sum(cntp_ref[...], axis=0)[:, None]
    mean = vsum_ref[...] / jnp.maximum(cnt, 1.0)
    vox = jnp.dot(mean, wv_ref[...], preferred_element_type=jnp.float32)
    vox = jnp.maximum(vox + bv_ref[...], 0.0)
    voxp_ref[...] = jnp.dot(vox, w1_ref[...],
                            preferred_element_type=jnp.float32)


def _s2(vsum, cntp, wv, bvr, w1):
    return pl.pallas_call(
        _s2_body,
        grid=(V_PAD // BV,),
        in_specs=[
            pl.BlockSpec((BV, H), lambda i: (i, 0)),
            pl.BlockSpec((NS, BV), lambda i: (0, i)),
            pl.BlockSpec((H, H), lambda i: (0, 0)),
            pl.BlockSpec((1, H), lambda i: (0, 0)),
            pl.BlockSpec((H, CS), lambda i: (0, 0)),
        ],
        out_specs=pl.BlockSpec((BV, CS), lambda i: (i, 0)),
        out_shape=jax.ShapeDtypeStruct((V_PAD, CS), jnp.float32),
    )(vsum, cntp, wv, bvr, w1)


def _s3_body(voxp_hbm, idx_hbm, voxg_hbm, idxbuf, gbuf, sem):
    c = lax.axis_index("c")
    s = lax.axis_index("s")
    wid = s * NC + c
    pltpu.sync_copy(idx_hbm.at[pl.ds(wid * C3, C3)], idxbuf)

    def _chunk(j, _):
        pltpu.async_copy(voxp_hbm.at[idxbuf.at[j]], gbuf, sem).wait()
        pltpu.sync_copy(gbuf,
                        voxg_hbm.at[pl.ds(wid * T3 + j * 128, 128), :])
        return 0
    lax.fori_loop(0, C3, _chunk, 0)


def _s3(voxp, idx2d):
    return pl.kernel(
        _s3_body,
        out_type=jax.ShapeDtypeStruct((N_PAD, CS), jnp.float32),
        mesh=_mesh,
        scratch_types=[
            pltpu.VMEM((C3, 128), jnp.int32),
            pltpu.VMEM((128, CS), jnp.float32),
            pltpu.SemaphoreType.DMA,
        ],
        compiler_params=_sc_params,
    )(voxp, idx2d)


def _s4_body(x_ref, voxg_ref, w0_ref, b0_ref, w1_ref, b1_ref,
             wc_ref, bc_ref, wc2_ref, bc2_ref,
             feats_ref, l1_ref, l2_ref):
    h = jnp.dot(x_ref[...], w0_ref[...], preferred_element_type=jnp.float32)
    h = jnp.maximum(h + b0_ref[...], 0.0)
    t = jnp.dot(h, w1_ref[...], preferred_element_type=jnp.float32)
    feats = jnp.maximum(t + voxg_ref[...] + b1_ref[...], 0.0)
    feats_ref[...] = feats
    l1_ref[...] = jnp.dot(feats, wc_ref[...],
                          preferred_element_type=jnp.float32) + bc_ref[...]
    l2_ref[...] = jnp.dot(feats, wc2_ref[...],
                          preferred_element_type=jnp.float32) + bc2_ref[...]


def _s4(x_pad, voxg, w0, b0r, w1, b1r, wc, bcr, wc2, bc2r):
    nblk = (N + BN - 1) // BN
    return pl.pallas_call(
        _s4_body,
        grid=(nblk,),
        in_specs=[
            pl.BlockSpec((BN, D_IN), lambda i: (i, 0)),
            pl.BlockSpec((BN, CS), lambda i: (i, 0)),
            pl.BlockSpec((D_IN, H), lambda i: (0, 0)),
            pl.BlockSpec((1, H), lambda i: (0, 0)),
            pl.BlockSpec((H, CS), lambda i: (0, 0)),
            pl.BlockSpec((1, CS), lambda i: (0, 0)),
            pl.BlockSpec((CS, NCLS), lambda i: (0, 0)),
            pl.BlockSpec((1, NCLS), lambda i: (0, 0)),
            pl.BlockSpec((CS, NCLS), lambda i: (0, 0)),
            pl.BlockSpec((1, NCLS), lambda i: (0, 0)),
        ],
        out_specs=[
            pl.BlockSpec((BN, CS), lambda i: (i, 0)),
            pl.BlockSpec((BN, NCLS), lambda i: (i, 0)),
            pl.BlockSpec((BN, NCLS), lambda i: (i, 0)),
        ],
        out_shape=[
            jax.ShapeDtypeStruct((N, CS), jnp.float32),
            jax.ShapeDtypeStruct((N, NCLS), jnp.float32),
            jax.ShapeDtypeStruct((N, NCLS), jnp.float32),
        ],
    )(x_pad, voxg, w0, b0r, w1, b1r, wc, bcr, wc2, bc2r)


@jax.jit
def kernel(pt_feats, voxel_idx, W0, b0, Wv, bv, W1, b1, Wc, bc, Wc2, bc2):
    x_pad = jnp.pad(pt_feats, ((0, N_PAD - N), (0, 0)))
    idx = voxel_idx.astype(jnp.int32)
    idx_pad = jnp.concatenate(
        [idx, jnp.broadcast_to(idx[-1:], (N_PAD - N,))])
    idx2d = idx_pad.reshape(N_PAD // 128, 128)

    b0r = b0.reshape(1, H)
    bvr = bv.reshape(1, H)
    b1r = b1.reshape(1, CS)
    bcr = bc.reshape(1, NCLS)
    bc2r = bc2.reshape(1, NCLS)

    h = _s0(x_pad, W0, b0r)
    vsum, cntp = _s1(h, idx2d)
    voxp = _s2(vsum, cntp, Wv, bvr, W1)
    voxg = _s3(voxp, idx2d)
    return _s4(x_pad, voxg, W0, b0r, W1, b1r, Wc, bcr, Wc2, bc2r)


# speedup vs baseline: 1.6448x; 1.6448x over previous
import jax
import jax.numpy as jnp
from jax import lax
from jax.experimental import pallas as pl
from jax.experimental.pallas import tpu as pltpu
from jax.experimental.pallas import tpu_sc as plsc

N = 100000
V = 20000
D_IN = 4
H = 256
CS = 96
NCLS = 19

N_PAD = 102400
V_PAD = 20480
BN0 = 2048
BN4 = 2048
BV = 512

NC, NS = 2, 16
T1 = N_PAD // NS
C1 = T1 // 128
T3 = N_PAD // (NC * NS)
C3 = T3 // 128
VROWS = V_PAD // NS

_mesh = plsc.VectorSubcoreMesh(core_axis_name="c", subcore_axis_name="s")
_sc_params = pltpu.CompilerParams(use_tc_tiling_on_sc=False,
                                  needs_layout_passes=False)


def _point_mlp(x_blk, w0_ref, b0_ref):
    acc = b0_ref[...]
    for k in range(D_IN):
        acc = acc + x_blk[:, k:k + 1] * w0_ref[k:k + 1, :]
    return jnp.maximum(acc, 0.0)


def _s0_body(x_ref, w0_ref, b0_ref, h_ref):
    i = pl.program_id(1)
    h = _point_mlp(x_ref[...], w0_ref, b0_ref)
    rows = i * BN0 + lax.broadcasted_iota(jnp.int32, (BN0, 1), 0)
    h_ref[0] = jnp.where(rows < N, h, 0.0)


def _s0(x, w0, b0r):
    return pl.pallas_call(
        _s0_body,
        grid=(2, N_PAD // BN0),
        in_specs=[
            pl.BlockSpec((BN0, D_IN),
                         lambda j, i: (jnp.minimum(i, (N - 1) // BN0), 0)),
            pl.BlockSpec((D_IN, 128), lambda j, i: (0, j)),
            pl.BlockSpec((1, 128), lambda j, i: (0, j)),
        ],
        out_specs=pl.BlockSpec((1, BN0, 128), lambda j, i: (j, i, 0)),
        out_shape=jax.ShapeDtypeStruct((2, N_PAD, 128), jnp.float32),
    )(x, w0, b0r)


def _s1_body(h_hbm, idx_hbm, vsum_hbm, cntp_hbm,
             acc, hbuf, idxbuf, zbuf, cntbuf):
    c = lax.axis_index("c")
    s = lax.axis_index("s")
    zero16 = jnp.zeros((16,), jnp.float32)

    def _zrow(i, _):
        for k in range(4):
            zbuf[i, pl.ds(k * 16, 16)] = zero16
        return 0
    lax.fori_loop(0, 128, _zrow, 0)

    def _zcnt(i, _):
        cntbuf[pl.ds(i * 16, 16)] = zero16
        return 0
    lax.fori_loop(0, V_PAD // 16, _zcnt, 0)

    pltpu.sync_copy(idx_hbm.at[pl.ds(s * C1, C1)], idxbuf)

    for p in range(2):
        colbase = p * 64
        def _zacc(i, _):
            pltpu.sync_copy(zbuf, acc.at[pl.ds(s * VROWS + i * 128, 128), :])
            return 0
        lax.fori_loop(0, VROWS // 128, _zacc, 0)
        plsc.subcore_barrier()

        def _chunk(j, _):
            rowstart = s * T1 + j * 128
            pltpu.sync_copy(
                h_hbm.at[c, pl.ds(rowstart, 128), pl.ds(colbase, 64)], hbuf)
            pltpu.sync_copy(hbuf, acc.at[idxbuf.at[j]], add=True)
            return 0
        lax.fori_loop(0, C1, _chunk, 0)
        plsc.subcore_barrier()

        pltpu.sync_copy(
            acc.at[pl.ds(s * VROWS, VROWS), :],
            vsum_hbm.at[c, pl.ds(s * VROWS, VROWS), pl.ds(colbase, 64)])

    @pl.when(c == 0)
    def _():
        ones = jnp.ones((16,), jnp.float32)
        iota = lax.iota(jnp.int32, 16)

        def _cchunk(j, _):
            def _cvec(k, _):
                gbase = s * T1 + j * 128 + k * 16
                iv = idxbuf[j, pl.ds(k * 16, 16)]
                mask = (gbase + iota) < N
                plsc.addupdate_scatter(cntbuf, [iv], ones, mask=mask)
                return 0
            lax.fori_loop(0, 8, _cvec, 0)
            return 0
        lax.fori_loop(0, C1, _cchunk, 0)
        pltpu.sync_copy(cntbuf, cntp_hbm.at[s])


def _s1(h3, idx2d):
    return pl.kernel(
        _s1_body,
        out_type=[
            jax.ShapeDtypeStruct((2, V_PAD, 128), jnp.float32),
            jax.ShapeDtypeStruct((NS, V_PAD), jnp.float32),
        ],
        mesh=_mesh,
        scratch_types=[
            pltpu.VMEM_SHARED((V_PAD, 64), jnp.float32),
            pltpu.VMEM((128, 64), jnp.float32),
            pltpu.VMEM((C1, 128), jnp.int32),
            pltpu.VMEM((128, 64), jnp.float32),
            pltpu.VMEM((V_PAD,), jnp.float32),
        ],
        compiler_params=_sc_params,
    )(h3, idx2d)


def _s2_body(vsum_ref, cntp_ref, wv_ref, bv_ref, w1_ref, voxp_ref):
    i = pl.program_id(0)
    cnt = jnp.sum(cntp_ref[:, pl.ds(i * BV, BV)], axis=0)[:, None]
    vs = jnp.concatenate([vsum_ref[0], vsum_ref[1]], axis=1)
    mean = vs / jnp.maximum(cnt, 1.0)
    vox = jnp.dot(mean, wv_ref[...], preferred_element_type=jnp.float32)
    vox = jnp.maximum(vox + bv_ref[...], 0.0)
    voxp = jnp.dot(vox, w1_ref[...], preferred_element_type=jnp.float32)
    voxp_ref[...] = jnp.concatenate(
        [voxp, jnp.zeros((BV, 128 - CS), jnp.float32)], axis=1)


def _s2(vsum, cntp, wv, bvr, w1):
    return pl.pallas_call(
        _s2_body,
        grid=(V_PAD // BV,),
        in_specs=[
            pl.BlockSpec((2, BV, 128), lambda i: (0, i, 0)),
            pl.BlockSpec((NS, V_PAD), lambda i: (0, 0)),
            pl.BlockSpec((H, H), lambda i: (0, 0)),
            pl.BlockSpec((1, H), lambda i: (0, 0)),
            pl.BlockSpec((H, CS), lambda i: (0, 0)),
        ],
        out_specs=pl.BlockSpec((BV, 128), lambda i: (i, 0)),
        out_shape=jax.ShapeDtypeStruct((V_PAD, 128), jnp.float32),
    )(vsum, cntp, wv, bvr, w1)


def _s3_body(voxp_hbm, idx_hbm, voxg_hbm, idxbuf, gbuf, sem):
    c = lax.axis_index("c")
    s = lax.axis_index("s")
    wid = s * NC + c
    pltpu.sync_copy(idx_hbm.at[pl.ds(wid * C3, C3)], idxbuf)

    def _chunk(j, _):
        pltpu.async_copy(voxp_hbm.at[idxbuf.at[j]], gbuf, sem).wait()
        pltpu.sync_copy(gbuf,
                        voxg_hbm.at[pl.ds(wid * T3 + j * 128, 128), :])
        return 0
    lax.fori_loop(0, C3, _chunk, 0)


def _s3(voxp, idx2d):
    return pl.kernel(
        _s3_body,
        out_type=jax.ShapeDtypeStruct((N_PAD, 128), jnp.float32),
        mesh=_mesh,
        scratch_types=[
            pltpu.VMEM((C3, 128), jnp.int32),
            pltpu.VMEM((128, 128), jnp.float32),
            pltpu.SemaphoreType.DMA,
        ],
        compiler_params=_sc_params,
    )(voxp, idx2d)


def _s4_body(x_ref, voxg_ref, w0_ref, b0_ref, w1_ref, b1_ref,
             wc_ref, bc_ref, wc2_ref, bc2_ref,
             feats_ref, l1_ref, l2_ref):
    h = _point_mlp(x_ref[...], w0_ref, b0_ref)
    t = jnp.dot(h, w1_ref[...], preferred_element_type=jnp.float32)
    feats = jnp.maximum(t + voxg_ref[:, :CS] + b1_ref[...], 0.0)
    feats_ref[...] = feats
    l1_ref[...] = jnp.dot(feats, wc_ref[...],
                          preferred_element_type=jnp.float32) + bc_ref[...]
    l2_ref[...] = jnp.dot(feats, wc2_ref[...],
                          preferred_element_type=jnp.float32) + bc2_ref[...]


def _s4(x, voxg, w0, b0r, w1, b1r, wc, bcr, wc2, bc2r):
    nblk = (N + BN4 - 1) // BN4
    return pl.pallas_call(
        _s4_body,
        grid=(nblk,),
        in_specs=[
            pl.BlockSpec((BN4, D_IN), lambda i: (i, 0)),
            pl.BlockSpec((BN4, 128), lambda i: (i, 0)),
            pl.BlockSpec((D_IN, H), lambda i: (0, 0)),
            pl.BlockSpec((1, H), lambda i: (0, 0)),
            pl.BlockSpec((H, CS), lambda i: (0, 0)),
            pl.BlockSpec((1, CS), lambda i: (0, 0)),
            pl.BlockSpec((CS, NCLS), lambda i: (0, 0)),
            pl.BlockSpec((1, NCLS), lambda i: (0, 0)),
            pl.BlockSpec((CS, NCLS), lambda i: (0, 0)),
            pl.BlockSpec((1, NCLS), lambda i: (0, 0)),
        ],
        out_specs=[
            pl.BlockSpec((BN4, CS), lambda i: (i, 0)),
            pl.BlockSpec((BN4, NCLS), lambda i: (i, 0)),
            pl.BlockSpec((BN4, NCLS), lambda i: (i, 0)),
        ],
        out_shape=[
            jax.ShapeDtypeStruct((N, CS), jnp.float32),
            jax.ShapeDtypeStruct((N, NCLS), jnp.float32),
            jax.ShapeDtypeStruct((N, NCLS), jnp.float32),
        ],
    )(x, voxg, w0, b0r, w1, b1r, wc, bcr, wc2, bc2r)


@jax.jit
def kernel(pt_feats, voxel_idx, W0, b0, Wv, bv, W1, b1, Wc, bc, Wc2, bc2):
    idx = voxel_idx.astype(jnp.int32)
    idx_pad = jnp.concatenate(
        [idx, jnp.broadcast_to(idx[-1:], (N_PAD - N,))])
    idx2d = idx_pad.reshape(N_PAD // 128, 128)

    b0r = b0.reshape(1, H)
    bvr = bv.reshape(1, H)
    b1r = b1.reshape(1, CS)
    bcr = bc.reshape(1, NCLS)
    bc2r = bc2.reshape(1, NCLS)

    h3 = _s0(pt_feats, W0, b0r)
    vsum, cntp = _s1(h3, idx2d)
    voxp = _s2(vsum, cntp, Wv, bvr, W1)
    voxg = _s3(voxp, idx2d)
    return _s4(pt_feats, voxg, W0, b0r, W1, b1r, Wc, bcr, Wc2, bc2r)


# speedup vs baseline: 2.0701x; 1.2585x over previous
import jax
import jax.numpy as jnp
from jax import lax
from jax.experimental import pallas as pl
from jax.experimental.pallas import tpu as pltpu
from jax.experimental.pallas import tpu_sc as plsc

N = 100000
V = 20000
D_IN = 4
H = 256
CS = 96
NCLS = 19

N_PAD = 102400
V_PAD = 20480
BN0 = 2048
BN4 = 2048
BV = 512

NC, NS = 2, 16
T1 = N_PAD // NS
C1 = T1 // 128
T3 = N_PAD // (NC * NS)
C3 = T3 // 128
VROWS = V_PAD // NS

_mesh = plsc.VectorSubcoreMesh(core_axis_name="c", subcore_axis_name="s")
_sc_params = pltpu.CompilerParams(use_tc_tiling_on_sc=False,
                                  needs_layout_passes=False)


def _point_mlp(x_blk, w0_ref, b0_ref):
    acc = b0_ref[...]
    for k in range(D_IN):
        acc = acc + x_blk[:, k:k + 1] * w0_ref[k:k + 1, :]
    return jnp.maximum(acc, 0.0)


def _s0_body(x_ref, w0_ref, b0_ref, h_ref):
    i = pl.program_id(0)
    h = _point_mlp(x_ref[...], w0_ref, b0_ref)
    rows = i * BN0 + lax.broadcasted_iota(jnp.int32, (BN0, 1), 0)
    h = jnp.where(rows < N, h, 0.0)
    h_ref[0] = h[:, :128]
    h_ref[1] = h[:, 128:]


def _s0(x, w0, b0r):
    return pl.pallas_call(
        _s0_body,
        grid=(N_PAD // BN0,),
        in_specs=[
            pl.BlockSpec((BN0, D_IN),
                         lambda i: (jnp.minimum(i, (N - 1) // BN0), 0)),
            pl.BlockSpec((D_IN, H), lambda i: (0, 0)),
            pl.BlockSpec((1, H), lambda i: (0, 0)),
        ],
        out_specs=pl.BlockSpec((2, BN0, 128), lambda i: (0, i, 0)),
        out_shape=jax.ShapeDtypeStruct((2, N_PAD, 128), jnp.float32),
    )(x, w0, b0r)


def _s1_body(h_hbm, idx_hbm, vsum_hbm, cntp_hbm,
             acc, hbuf0, hbuf1, idxbuf, zbuf, cntbuf, rsem0, rsem1):
    c = lax.axis_index("c")
    s = lax.axis_index("s")
    zero16 = jnp.zeros((16,), jnp.float32)
    bufs = (hbuf0, hbuf1)
    sems = (rsem0, rsem1)

    def _zrow(i, _):
        for k in range(4):
            zbuf[i, pl.ds(k * 16, 16)] = zero16
        return 0
    lax.fori_loop(0, 64, _zrow, 0)

    def _zcnt(i, _):
        cntbuf[pl.ds(i * 16, 16)] = zero16
        return 0
    lax.fori_loop(0, V_PAD // 16, _zcnt, 0)

    pltpu.sync_copy(idx_hbm.at[pl.ds(s * C1, C1)], idxbuf)

    for p in range(2):
        colbase = p * 64

        def _hsrc(j):
            return h_hbm.at[c, pl.ds(s * T1 + j * 128, 128),
                            pl.ds(colbase, 64)]

        def _zacc(i, _):
            pltpu.sync_copy(zbuf, acc.at[pl.ds(s * VROWS + i * 64, 64), :])
            return 0
        lax.fori_loop(0, VROWS // 64, _zacc, 0)
        plsc.subcore_barrier()

        pltpu.async_copy(_hsrc(0), hbuf0, rsem0)
        pltpu.async_copy(_hsrc(1), hbuf1, rsem1)

        def _pipe(j2, _):
            for b in range(2):
                j = 2 * j2 + b
                pltpu.make_async_copy(_hsrc(j), bufs[b], sems[b]).wait()
                pltpu.sync_copy(bufs[b], acc.at[idxbuf.at[j]], add=True)

                @pl.when(j + 2 < C1)
                def _():
                    pltpu.async_copy(_hsrc(j + 2), bufs[b], sems[b])
            return 0
        lax.fori_loop(0, C1 // 2, _pipe, 0)
        plsc.subcore_barrier()

        pltpu.sync_copy(
            acc.at[pl.ds(s * VROWS, VROWS), :],
            vsum_hbm.at[c, pl.ds(s * VROWS, VROWS), pl.ds(colbase, 64)])

    @pl.when(c == 0)
    def _():
        ones = jnp.ones((16,), jnp.float32)
        iota = lax.iota(jnp.int32, 16)

        def _cchunk(j, _):
            def _cvec(k, _):
                gbase = s * T1 + j * 128 + k * 16
                iv = idxbuf[j, pl.ds(k * 16, 16)]
                mask = (gbase + iota) < N
                plsc.addupdate_scatter(cntbuf, [iv], ones, mask=mask)
                return 0
            lax.fori_loop(0, 8, _cvec, 0)
            return 0
        lax.fori_loop(0, C1, _cchunk, 0)
        pltpu.sync_copy(cntbuf, cntp_hbm.at[s])


def _s1(h3, idx2d):
    return pl.kernel(
        _s1_body,
        out_type=[
            jax.ShapeDtypeStruct((2, V_PAD, 128), jnp.float32),
            jax.ShapeDtypeStruct((NS, V_PAD), jnp.float32),
        ],
        mesh=_mesh,
        scratch_types=[
            pltpu.VMEM_SHARED((V_PAD, 64), jnp.float32),
            pltpu.VMEM((128, 64), jnp.float32),
            pltpu.VMEM((128, 64), jnp.float32),
            pltpu.VMEM((C1, 128), jnp.int32),
            pltpu.VMEM((64, 64), jnp.float32),
            pltpu.VMEM((V_PAD,), jnp.float32),
            pltpu.SemaphoreType.DMA,
            pltpu.SemaphoreType.DMA,
        ],
        compiler_params=_sc_params,
    )(h3, idx2d)


def _s2_body(vsum_ref, cntp_ref, wv_ref, bv_ref, w1_ref, voxp_ref):
    i = pl.program_id(0)
    cnt = jnp.sum(cntp_ref[:, pl.ds(i * BV, BV)], axis=0)[:, None]
    vs = jnp.concatenate([vsum_ref[0], vsum_ref[1]], axis=1)
    mean = vs / jnp.maximum(cnt, 1.0)
    vox = jnp.dot(mean, wv_ref[...], preferred_element_type=jnp.float32)
    vox = jnp.maximum(vox + bv_ref[...], 0.0)
    voxp_ref[...] = jnp.dot(vox, w1_ref[...],
                            preferred_element_type=jnp.float32)


def _s2(vsum, cntp, wv, bvr, w1):
    return pl.pallas_call(
        _s2_body,
        grid=(V_PAD // BV,),
        in_specs=[
            pl.BlockSpec((2, BV, 128), lambda i: (0, i, 0)),
            pl.BlockSpec((NS, V_PAD), lambda i: (0, 0)),
            pl.BlockSpec((H, H), lambda i: (0, 0)),
            pl.BlockSpec((1, H), lambda i: (0, 0)),
            pl.BlockSpec((H, CS), lambda i: (0, 0)),
        ],
        out_specs=pl.BlockSpec((BV, CS), lambda i: (i, 0)),
        out_shape=jax.ShapeDtypeStruct((V_PAD, CS), jnp.float32),
    )(vsum, cntp, wv, bvr, w1)


def _s3_body(voxp_hbm, idx_hbm, voxg_hbm, idxbuf, gbuf0, gbuf1, gsem0, gsem1):
    c = lax.axis_index("c")
    s = lax.axis_index("s")
    wid = s * NC + c
    bufs = (gbuf0, gbuf1)
    sems = (gsem0, gsem1)
    pltpu.sync_copy(idx_hbm.at[pl.ds(wid * C3, C3)], idxbuf)

    pltpu.async_copy(voxp_hbm.at[idxbuf.at[0]], gbuf0, gsem0)
    pltpu.async_copy(voxp_hbm.at[idxbuf.at[1]], gbuf1, gsem1)

    def _pipe(j2, _):
        for b in range(2):
            j = 2 * j2 + b

            @pl.when(j < C3)
            def _():
                pltpu.make_async_copy(
                    voxp_hbm.at[idxbuf.at[j]], bufs[b], sems[b]).wait()
                pltpu.sync_copy(
                    bufs[b],
                    voxg_hbm.at[pl.ds(wid * T3 + j * 128, 128),
                                pl.ds(0, CS)])

                @pl.when(j + 2 < C3)
                def _():
                    pltpu.async_copy(
                        voxp_hbm.at[idxbuf.at[j + 2]], bufs[b], sems[b])
        return 0
    lax.fori_loop(0, (C3 + 1) // 2, _pipe, 0)


def _s3(voxp, idx2d):
    return pl.kernel(
        _s3_body,
        out_type=jax.ShapeDtypeStruct((N_PAD, 128), jnp.float32),
        mesh=_mesh,
        scratch_types=[
            pltpu.VMEM((C3, 128), jnp.int32),
            pltpu.VMEM((128, CS), jnp.float32),
            pltpu.VMEM((128, CS), jnp.float32),
            pltpu.SemaphoreType.DMA,
            pltpu.SemaphoreType.DMA,
        ],
        compiler_params=_sc_params,
    )(voxp, idx2d)


def _s4_body(x_ref, voxg_ref, w0_ref, b0_ref, w1_ref, b1_ref,
             wc_ref, bc_ref, wc2_ref, bc2_ref,
             feats_ref, l1_ref, l2_ref):
    h = _point_mlp(x_ref[...], w0_ref, b0_ref)
    t = jnp.dot(h, w1_ref[...], preferred_element_type=jnp.float32)
    feats = jnp.maximum(t + voxg_ref[:, :CS] + b1_ref[...], 0.0)
    feats_ref[...] = feats
    l1_ref[...] = jnp.dot(feats, wc_ref[...],
                          preferred_element_type=jnp.float32) + bc_ref[...]
    l2_ref[...] = jnp.dot(feats, wc2_ref[...],
                          preferred_element_type=jnp.float32) + bc2_ref[...]


def _s4(x, voxg, w0, b0r, w1, b1r, wc, bcr, wc2, bc2r):
    nblk = (N + BN4 - 1) // BN4
    return pl.pallas_call(
        _s4_body,
        grid=(nblk,),
        in_specs=[
            pl.BlockSpec((BN4, D_IN), lambda i: (i, 0)),
            pl.BlockSpec((BN4, 128), lambda i: (i, 0)),
            pl.BlockSpec((D_IN, H), lambda i: (0, 0)),
            pl.BlockSpec((1, H), lambda i: (0, 0)),
            pl.BlockSpec((H, CS), lambda i: (0, 0)),
            pl.BlockSpec((1, CS), lambda i: (0, 0)),
            pl.BlockSpec((CS, NCLS), lambda i: (0, 0)),
            pl.BlockSpec((1, NCLS), lambda i: (0, 0)),
            pl.BlockSpec((CS, NCLS), lambda i: (0, 0)),
            pl.BlockSpec((1, NCLS), lambda i: (0, 0)),
        ],
        out_specs=[
            pl.BlockSpec((BN4, CS), lambda i: (i, 0)),
            pl.BlockSpec((BN4, NCLS), lambda i: (i, 0)),
            pl.BlockSpec((BN4, NCLS), lambda i: (i, 0)),
        ],
        out_shape=[
            jax.ShapeDtypeStruct((N, CS), jnp.float32),
            jax.ShapeDtypeStruct((N, NCLS), jnp.float32),
            jax.ShapeDtypeStruct((N, NCLS), jnp.float32),
        ],
    )(x, voxg, w0, b0r, w1, b1r, wc, bcr, wc2, bc2r)


@jax.jit
def kernel(pt_feats, voxel_idx, W0, b0, Wv, bv, W1, b1, Wc, bc, Wc2, bc2):
    idx = voxel_idx.astype(jnp.int32)
    idx_pad = jnp.concatenate(
        [idx, jnp.broadcast_to(idx[-1:], (N_PAD - N,))])
    idx2d = idx_pad.reshape(N_PAD // 128, 128)

    b0r = b0.reshape(1, H)
    bvr = bv.reshape(1, H)
    b1r = b1.reshape(1, CS)
    bcr = bc.reshape(1, NCLS)
    bc2r = bc2.reshape(1, NCLS)

    h3 = _s0(pt_feats, W0, b0r)
    vsum, cntp = _s1(h3, idx2d)
    voxp = _s2(vsum, cntp, Wv, bvr, W1)
    voxg = _s3(voxp, idx2d)
    return _s4(pt_feats, voxg, W0, b0r, W1, b1r, Wc, bcr, Wc2, bc2r)


# speedup vs baseline: 3.0214x; 1.4596x over previous
import jax
import jax.numpy as jnp
from jax import lax
from jax.experimental import pallas as pl
from jax.experimental.pallas import tpu as pltpu
from jax.experimental.pallas import tpu_sc as plsc

N = 100000
V = 20000
D_IN = 4
H = 256
CS = 96
NCLS = 19

N_PAD = 102400
V_PAD = 20480
BN0 = 2048
BN4 = 2048
BV = 512

NC, NS = 2, 16
T1 = N_PAD // NS
C1 = T1 // 128
T3 = N_PAD // (NC * NS)
C3 = T3 // 128
VROWS = V_PAD // NS

_mesh = plsc.VectorSubcoreMesh(core_axis_name="c", subcore_axis_name="s")
_sc_params = pltpu.CompilerParams(use_tc_tiling_on_sc=False,
                                  needs_layout_passes=False)


def _s0_body(xt_ref, w0_ref, b0_ref, h_ref):
    i = pl.program_id(0)
    h = lax.dot_general(xt_ref[...], w0_ref[...],
                        (((0,), (0,)), ((), ())),
                        preferred_element_type=jnp.float32)
    h = jnp.maximum(h + b0_ref[...], 0.0)
    rows = i * BN0 + lax.broadcasted_iota(jnp.int32, (BN0, 1), 0)
    h = jnp.where(rows < N, h, 0.0)
    h_ref[0] = h[:, :128]
    h_ref[1] = h[:, 128:]


def _s0(xt, w0, b0r):
    return pl.pallas_call(
        _s0_body,
        grid=(N_PAD // BN0,),
        in_specs=[
            pl.BlockSpec((D_IN, BN0), lambda i: (0, i)),
            pl.BlockSpec((D_IN, H), lambda i: (0, 0)),
            pl.BlockSpec((1, H), lambda i: (0, 0)),
        ],
        out_specs=pl.BlockSpec((2, BN0, 128), lambda i: (0, i, 0)),
        out_shape=jax.ShapeDtypeStruct((2, N_PAD, 128), jnp.float32),
    )(xt, w0, b0r)


def _s1_body(h_hbm, idx_hbm, vsum_hbm, cntp_hbm,
             acc, hbuf0, hbuf1, idxbuf, zbuf, cntbuf, rsem0, rsem1):
    c = lax.axis_index("c")
    s = lax.axis_index("s")
    zero16 = jnp.zeros((16,), jnp.float32)
    bufs = (hbuf0, hbuf1)
    sems = (rsem0, rsem1)

    def _zrow(i, _):
        for k in range(4):
            zbuf[i, pl.ds(k * 16, 16)] = zero16
        return 0
    lax.fori_loop(0, 64, _zrow, 0)

    def _zcnt(i, _):
        cntbuf[pl.ds(i * 16, 16)] = zero16
        return 0
    lax.fori_loop(0, V_PAD // 16, _zcnt, 0)

    pltpu.sync_copy(idx_hbm.at[pl.ds(s * C1, C1)], idxbuf)

    for p in range(2):
        colbase = p * 64

        def _hsrc(j):
            return h_hbm.at[c, pl.ds(s * T1 + j * 128, 128),
                            pl.ds(colbase, 64)]

        def _zacc(i, _):
            pltpu.sync_copy(zbuf, acc.at[pl.ds(s * VROWS + i * 64, 64), :])
            return 0
        lax.fori_loop(0, VROWS // 64, _zacc, 0)
        plsc.subcore_barrier()

        pltpu.async_copy(_hsrc(0), hbuf0, rsem0)
        pltpu.async_copy(_hsrc(1), hbuf1, rsem1)

        def _pipe(j2, _):
            for b in range(2):
                j = 2 * j2 + b
                pltpu.make_async_copy(_hsrc(j), bufs[b], sems[b]).wait()
                pltpu.sync_copy(bufs[b], acc.at[idxbuf.at[j]], add=True)

                @pl.when(j + 2 < C1)
                def _():
                    pltpu.async_copy(_hsrc(j + 2), bufs[b], sems[b])
            return 0
        lax.fori_loop(0, C1 // 2, _pipe, 0)
        plsc.subcore_barrier()

        pltpu.sync_copy(
            acc.at[pl.ds(s * VROWS, VROWS), :],
            vsum_hbm.at[c, pl.ds(s * VROWS, VROWS), pl.ds(colbase, 64)])

    @pl.when(c == 0)
    def _():
        ones = jnp.ones((16,), jnp.float32)
        iota = lax.iota(jnp.int32, 16)

        def _cchunk(j, _):
            def _cvec(k, _):
                gbase = s * T1 + j * 128 + k * 16
                iv = idxbuf[j, pl.ds(k * 16, 16)]
                mask = (gbase + iota) < N
                plsc.addupdate_scatter(cntbuf, [iv], ones, mask=mask)
                return 0
            lax.fori_loop(0, 8, _cvec, 0)
            return 0
        lax.fori_loop(0, C1, _cchunk, 0)
        pltpu.sync_copy(cntbuf, cntp_hbm.at[s])


def _s1(h3, idx2d):
    return pl.kernel(
        _s1_body,
        out_type=[
            jax.ShapeDtypeStruct((2, V_PAD, 128), jnp.float32),
            jax.ShapeDtypeStruct((NS, V_PAD), jnp.float32),
        ],
        mesh=_mesh,
        scratch_types=[
            pltpu.VMEM_SHARED((V_PAD, 64), jnp.float32),
            pltpu.VMEM((128, 64), jnp.float32),
            pltpu.VMEM((128, 64), jnp.float32),
            pltpu.VMEM((C1, 128), jnp.int32),
            pltpu.VMEM((64, 64), jnp.float32),
            pltpu.VMEM((V_PAD,), jnp.float32),
            pltpu.SemaphoreType.DMA,
            pltpu.SemaphoreType.DMA,
        ],
        compiler_params=_sc_params,
    )(h3, idx2d)


def _s2_body(vsum_ref, cntp_ref, wv_ref, bv_ref, w1_ref, voxp_ref):
    i = pl.program_id(0)
    cnt = jnp.sum(cntp_ref[:, pl.ds(i * BV, BV)], axis=0)[:, None]
    vs = jnp.concatenate([vsum_ref[0], vsum_ref[1]], axis=1)
    mean = vs / jnp.maximum(cnt, 1.0)
    vox = jnp.dot(mean, wv_ref[...], preferred_element_type=jnp.float32)
    vox = jnp.maximum(vox + bv_ref[...], 0.0)
    voxp_ref[...] = jnp.dot(vox, w1_ref[...],
                            preferred_element_type=jnp.float32)


def _s2(vsum, cntp, wv, bvr, w1):
    return pl.pallas_call(
        _s2_body,
        grid=(V_PAD // BV,),
        in_specs=[
            pl.BlockSpec((2, BV, 128), lambda i: (0, i, 0)),
            pl.BlockSpec((NS, V_PAD), lambda i: (0, 0)),
            pl.BlockSpec((H, H), lambda i: (0, 0)),
            pl.BlockSpec((1, H), lambda i: (0, 0)),
            pl.BlockSpec((H, CS), lambda i: (0, 0)),
        ],
        out_specs=pl.BlockSpec((BV, CS), lambda i: (i, 0)),
        out_shape=jax.ShapeDtypeStruct((V_PAD, CS), jnp.float32),
    )(vsum, cntp, wv, bvr, w1)


def _s3_body(voxp_hbm, idx_hbm, voxg_hbm, idxbuf, gbuf0, gbuf1, gsem0, gsem1):
    c = lax.axis_index("c")
    s = lax.axis_index("s")
    wid = s * NC + c
    bufs = (gbuf0, gbuf1)
    sems = (gsem0, gsem1)
    pltpu.sync_copy(idx_hbm.at[pl.ds(wid * C3, C3)], idxbuf)

    pltpu.async_copy(voxp_hbm.at[idxbuf.at[0]], gbuf0, gsem0)
    pltpu.async_copy(voxp_hbm.at[idxbuf.at[1]], gbuf1, gsem1)

    def _pipe(j2, _):
        for b in range(2):
            j = 2 * j2 + b

            @pl.when(j < C3)
            def _():
                pltpu.make_async_copy(
                    voxp_hbm.at[idxbuf.at[j]], bufs[b], sems[b]).wait()
                pltpu.sync_copy(
                    bufs[b],
                    voxg_hbm.at[pl.ds(wid * T3 + j * 128, 128),
                                pl.ds(0, CS)])

                @pl.when(j + 2 < C3)
                def _():
                    pltpu.async_copy(
                        voxp_hbm.at[idxbuf.at[j + 2]], bufs[b], sems[b])
        return 0
    lax.fori_loop(0, (C3 + 1) // 2, _pipe, 0)


def _s3(voxp, idx2d):
    return pl.kernel(
        _s3_body,
        out_type=jax.ShapeDtypeStruct((N_PAD, 128), jnp.float32),
        mesh=_mesh,
        scratch_types=[
            pltpu.VMEM((C3, 128), jnp.int32),
            pltpu.VMEM((128, CS), jnp.float32),
            pltpu.VMEM((128, CS), jnp.float32),
            pltpu.SemaphoreType.DMA,
            pltpu.SemaphoreType.DMA,
        ],
        compiler_params=_sc_params,
    )(voxp, idx2d)


def _s4_body(xt_ref, voxg_ref, w0t_ref, b0c_ref, w1t_ref, b1c_ref,
             wct_ref, bcc_ref, wc2t_ref, bc2c_ref,
             feats_ref, l1_ref, l2_ref):
    ht = b0c_ref[...]
    for k in range(D_IN):
        ht = ht + w0t_ref[:, k:k + 1] * xt_ref[k:k + 1, :]
    ht = jnp.maximum(ht, 0.0)
    vgt = voxg_ref[...].T
    t = jnp.dot(w1t_ref[...], ht, preferred_element_type=jnp.float32)
    feats = jnp.maximum(t + vgt[:CS, :] + b1c_ref[...], 0.0)
    feats_ref[...] = feats
    l1_ref[...] = jnp.dot(wct_ref[...], feats,
                          preferred_element_type=jnp.float32) + bcc_ref[...]
    l2_ref[...] = jnp.dot(wc2t_ref[...], feats,
                          preferred_element_type=jnp.float32) + bc2c_ref[...]


def _s4(xt, voxg, w0t, b0c, w1t, b1c, wct, bcc, wc2t, bc2c):
    nblk = (N + BN4 - 1) // BN4
    return pl.pallas_call(
        _s4_body,
        grid=(nblk,),
        in_specs=[
            pl.BlockSpec((D_IN, BN4), lambda i: (0, i)),
            pl.BlockSpec((BN4, 128), lambda i: (i, 0)),
            pl.BlockSpec((H, D_IN), lambda i: (0, 0)),
            pl.BlockSpec((H, 1), lambda i: (0, 0)),
            pl.BlockSpec((CS, H), lambda i: (0, 0)),
            pl.BlockSpec((CS, 1), lambda i: (0, 0)),
            pl.BlockSpec((NCLS, CS), lambda i: (0, 0)),
            pl.BlockSpec((NCLS, 1), lambda i: (0, 0)),
            pl.BlockSpec((NCLS, CS), lambda i: (0, 0)),
            pl.BlockSpec((NCLS, 1), lambda i: (0, 0)),
        ],
        out_specs=[
            pl.BlockSpec((CS, BN4), lambda i: (0, i)),
            pl.BlockSpec((NCLS, BN4), lambda i: (0, i)),
            pl.BlockSpec((NCLS, BN4), lambda i: (0, i)),
        ],
        out_shape=[
            jax.ShapeDtypeStruct((CS, N), jnp.float32),
            jax.ShapeDtypeStruct((NCLS, N), jnp.float32),
            jax.ShapeDtypeStruct((NCLS, N), jnp.float32),
        ],
    )(xt, voxg, w0t, b0c, w1t, b1c, wct, bcc, wc2t, bc2c)


@jax.jit
def kernel(pt_feats, voxel_idx, W0, b0, Wv, bv, W1, b1, Wc, bc, Wc2, bc2):
    idx = voxel_idx.astype(jnp.int32)
    idx_pad = jnp.concatenate(
        [idx, jnp.broadcast_to(idx[-1:], (N_PAD - N,))])
    idx2d = idx_pad.reshape(N_PAD // 128, 128)

    b0r = b0.reshape(1, H)
    bvr = bv.reshape(1, H)

    xt = jnp.pad(pt_feats.T, ((0, 0), (0, N_PAD - N)))
    h3 = _s0(xt, W0, b0r)
    vsum, cntp = _s1(h3, idx2d)
    voxp = _s2(vsum, cntp, Wv, bvr, W1)
    voxg = _s3(voxp, idx2d)
    ft, l1t, l2t = _s4(xt, voxg, W0.T, b0.reshape(H, 1),
                       W1.T, b1.reshape(CS, 1),
                       Wc.T, bc.reshape(NCLS, 1),
                       Wc2.T, bc2.reshape(NCLS, 1))
    return ft.T, l1t.T, l2t.T


# speedup vs baseline: 3.1546x; 1.0441x over previous
import jax
import jax.numpy as jnp
from jax import lax
from jax.experimental import pallas as pl
from jax.experimental.pallas import tpu as pltpu
from jax.experimental.pallas import tpu_sc as plsc

N = 100000
V = 20000
D_IN = 4
H = 256
CS = 96
NCLS = 19

N_PAD = 102400
V_PAD = 20480
BN0 = 2048
BN4 = 2048
BV = 512

NC, NS = 2, 16
T1 = N_PAD // NS
C1 = T1 // 128
T3 = N_PAD // (NC * NS)
C3 = T3 // 128
VROWS = V_PAD // NS

_mesh = plsc.VectorSubcoreMesh(core_axis_name="c", subcore_axis_name="s")
_sc_params = pltpu.CompilerParams(use_tc_tiling_on_sc=False,
                                  needs_layout_passes=False)


def _s0_body(xt_ref, w0_ref, b0_ref, h_ref):
    i = pl.program_id(0)
    h = lax.dot_general(xt_ref[...], w0_ref[...],
                        (((0,), (0,)), ((), ())),
                        preferred_element_type=jnp.float32)
    h = jnp.maximum(h + b0_ref[...], 0.0)
    rows = i * BN0 + lax.broadcasted_iota(jnp.int32, (BN0, 1), 0)
    h = jnp.where(rows < N, h, 0.0)
    h_ref[0] = h[:, :128]
    h_ref[1] = h[:, 128:]


def _s0(xt, w0, b0r):
    return pl.pallas_call(
        _s0_body,
        grid=(N_PAD // BN0,),
        in_specs=[
            pl.BlockSpec((D_IN, BN0), lambda i: (0, i)),
            pl.BlockSpec((D_IN, H), lambda i: (0, 0)),
            pl.BlockSpec((1, H), lambda i: (0, 0)),
        ],
        out_specs=pl.BlockSpec((2, BN0, 128), lambda i: (0, i, 0)),
        out_shape=jax.ShapeDtypeStruct((2, N_PAD, 128), jnp.float32),
    )(xt, w0, b0r)


def _scnt_body(idx_hbm, cntp_hbm, idxbuf, cntbuf):
    c = lax.axis_index("c")
    s = lax.axis_index("s")
    wid = s * NC + c
    zero16 = jnp.zeros((16,), jnp.float32)
    ones = jnp.ones((16,), jnp.float32)
    iota = lax.iota(jnp.int32, 16)

    def _zcnt(i, _):
        cntbuf[pl.ds(i * 16, 16)] = zero16
        return 0
    lax.fori_loop(0, V_PAD // 16, _zcnt, 0)
    pltpu.sync_copy(idx_hbm.at[pl.ds(wid * C3, C3)], idxbuf)

    def _cchunk(j, _):
        def _cvec(k, _):
            gbase = wid * T3 + j * 128 + k * 16
            iv = idxbuf[j, pl.ds(k * 16, 16)]
            mask = (gbase + iota) < N
            plsc.addupdate_scatter(cntbuf, [iv], ones, mask=mask)
            return 0
        lax.fori_loop(0, 8, _cvec, 0)
        return 0
    lax.fori_loop(0, C3, _cchunk, 0)
    pltpu.sync_copy(cntbuf, cntp_hbm.at[wid])


def _scnt(idx2d):
    return pl.kernel(
        _scnt_body,
        out_type=jax.ShapeDtypeStruct((NC * NS, V_PAD), jnp.float32),
        mesh=_mesh,
        scratch_types=[
            pltpu.VMEM((C3, 128), jnp.int32),
            pltpu.VMEM((V_PAD,), jnp.float32),
        ],
        compiler_params=_sc_params,
    )(idx2d)


def _s1_body(h_hbm, idx_hbm, vsum_hbm,
             acc, hbuf0, hbuf1, idxbuf, zbuf, rsem0, rsem1):
    c = lax.axis_index("c")
    s = lax.axis_index("s")
    zero16 = jnp.zeros((16,), jnp.float32)
    bufs = (hbuf0, hbuf1)
    sems = (rsem0, rsem1)

    def _zrow(i, _):
        for k in range(4):
            zbuf[i, pl.ds(k * 16, 16)] = zero16
        return 0
    lax.fori_loop(0, 64, _zrow, 0)

    pltpu.sync_copy(idx_hbm.at[pl.ds(s * C1, C1)], idxbuf)

    for p in range(2):
        colbase = p * 64

        def _hsrc(j):
            return h_hbm.at[c, pl.ds(s * T1 + j * 128, 128),
                            pl.ds(colbase, 64)]

        def _zacc(i, _):
            pltpu.sync_copy(zbuf, acc.at[pl.ds(s * VROWS + i * 64, 64), :])
            return 0
        lax.fori_loop(0, VROWS // 64, _zacc, 0)
        plsc.subcore_barrier()

        pltpu.async_copy(_hsrc(0), hbuf0, rsem0)
        pltpu.async_copy(_hsrc(1), hbuf1, rsem1)

        def _pipe(j2, _):
            for b in range(2):
                j = 2 * j2 + b
                pltpu.make_async_copy(_hsrc(j), bufs[b], sems[b]).wait()
                pltpu.sync_copy(bufs[b], acc.at[idxbuf.at[j]], add=True)

                @pl.when(j + 2 < C1)
                def _():
                    pltpu.async_copy(_hsrc(j + 2), bufs[b], sems[b])
            return 0
        lax.fori_loop(0, C1 // 2, _pipe, 0)
        plsc.subcore_barrier()

        pltpu.sync_copy(
            acc.at[pl.ds(s * VROWS, VROWS), :],
            vsum_hbm.at[c, pl.ds(s * VROWS, VROWS), pl.ds(colbase, 64)])


def _s1(h3, idx2d):
    return pl.kernel(
        _s1_body,
        out_type=jax.ShapeDtypeStruct((2, V_PAD, 128), jnp.float32),
        mesh=_mesh,
        scratch_types=[
            pltpu.VMEM_SHARED((V_PAD, 64), jnp.float32),
            pltpu.VMEM((128, 64), jnp.float32),
            pltpu.VMEM((128, 64), jnp.float32),
            pltpu.VMEM((C1, 128), jnp.int32),
            pltpu.VMEM((64, 64), jnp.float32),
            pltpu.SemaphoreType.DMA,
            pltpu.SemaphoreType.DMA,
        ],
        compiler_params=_sc_params,
    )(h3, idx2d)


def _s2_body(vsum_ref, cntp_ref, wv_ref, bv_ref, w1_ref, voxp_ref):
    i = pl.program_id(0)
    cnt = jnp.sum(cntp_ref[:, pl.ds(i * BV, BV)], axis=0)[:, None]
    vs = jnp.concatenate([vsum_ref[0], vsum_ref[1]], axis=1)
    mean = vs / jnp.maximum(cnt, 1.0)
    vox = jnp.dot(mean, wv_ref[...], preferred_element_type=jnp.float32)
    vox = jnp.maximum(vox + bv_ref[...], 0.0)
    voxp_ref[...] = jnp.dot(vox, w1_ref[...],
                            preferred_element_type=jnp.float32)


def _s2(vsum, cntp, wv, bvr, w1):
    return pl.pallas_call(
        _s2_body,
        grid=(V_PAD // BV,),
        in_specs=[
            pl.BlockSpec((2, BV, 128), lambda i: (0, i, 0)),
            pl.BlockSpec((NC * NS, V_PAD), lambda i: (0, 0)),
            pl.BlockSpec((H, H), lambda i: (0, 0)),
            pl.BlockSpec((1, H), lambda i: (0, 0)),
            pl.BlockSpec((H, CS), lambda i: (0, 0)),
        ],
        out_specs=pl.BlockSpec((BV, CS), lambda i: (i, 0)),
        out_shape=jax.ShapeDtypeStruct((V_PAD, CS), jnp.float32),
    )(vsum, cntp, wv, bvr, w1)


_NB3 = 4


def _s3_body(voxp_hbm, idx_hbm, voxg_hbm, idxbuf,
             gbuf0, gbuf1, gbuf2, gbuf3,
             gsem0, gsem1, gsem2, gsem3,
             wsem0, wsem1, wsem2, wsem3):
    c = lax.axis_index("c")
    s = lax.axis_index("s")
    wid = s * NC + c
    bufs = (gbuf0, gbuf1, gbuf2, gbuf3)
    gsems = (gsem0, gsem1, gsem2, gsem3)
    wsems = (wsem0, wsem1, wsem2, wsem3)
    pltpu.sync_copy(idx_hbm.at[pl.ds(wid * C3, C3)], idxbuf)

    def _dst(j):
        return voxg_hbm.at[pl.ds(wid * T3 + j * 128, 128), pl.ds(0, CS)]

    for b in range(_NB3):
        pltpu.async_copy(voxp_hbm.at[idxbuf.at[b]], bufs[b], gsems[b])

    def _pipe(j2, _):
        for b in range(_NB3):
            j = _NB3 * j2 + b

            @pl.when(j < C3)
            def _():
                pltpu.make_async_copy(
                    voxp_hbm.at[idxbuf.at[j]], bufs[b], gsems[b]).wait()
                pltpu.async_copy(bufs[b], _dst(j), wsems[b])

                @pl.when(j + _NB3 < C3)
                def _():
                    pltpu.make_async_copy(bufs[b], _dst(j), wsems[b]).wait()
                    pltpu.async_copy(
                        voxp_hbm.at[idxbuf.at[j + _NB3]], bufs[b], gsems[b])
        return 0
    lax.fori_loop(0, (C3 + _NB3 - 1) // _NB3, _pipe, 0)

    for b in range(_NB3):
        j_last = ((C3 - 1 - b) // _NB3) * _NB3 + b
        pltpu.make_async_copy(bufs[b], _dst(j_last), wsems[b]).wait()


def _s3(voxp, idx2d):
    return pl.kernel(
        _s3_body,
        out_type=jax.ShapeDtypeStruct((N_PAD, 128), jnp.float32),
        mesh=_mesh,
        scratch_types=[
            pltpu.VMEM((C3, 128), jnp.int32),
            pltpu.VMEM((128, CS), jnp.float32),
            pltpu.VMEM((128, CS), jnp.float32),
            pltpu.VMEM((128, CS), jnp.float32),
            pltpu.VMEM((128, CS), jnp.float32),
            pltpu.SemaphoreType.DMA,
            pltpu.SemaphoreType.DMA,
            pltpu.SemaphoreType.DMA,
            pltpu.SemaphoreType.DMA,
            pltpu.SemaphoreType.DMA,
            pltpu.SemaphoreType.DMA,
            pltpu.SemaphoreType.DMA,
            pltpu.SemaphoreType.DMA,
        ],
        compiler_params=_sc_params,
    )(voxp, idx2d)


def _s4_body(xt_ref, voxg_ref, w0t_ref, b0c_ref, w1t_ref, b1c_ref,
             wct_ref, bcc_ref, wc2t_ref, bc2c_ref,
             feats_ref, l1_ref, l2_ref):
    ht = b0c_ref[...]
    for k in range(D_IN):
        ht = ht + w0t_ref[:, k:k + 1] * xt_ref[k:k + 1, :]
    ht = jnp.maximum(ht, 0.0)
    vgt = voxg_ref[...].T
    t = jnp.dot(w1t_ref[...], ht, preferred_element_type=jnp.float32)
    feats = jnp.maximum(t + vgt[:CS, :] + b1c_ref[...], 0.0)
    feats_ref[...] = feats
    l1_ref[...] = jnp.dot(wct_ref[...], feats,
                          preferred_element_type=jnp.float32) + bcc_ref[...]
    l2_ref[...] = jnp.dot(wc2t_ref[...], feats,
                          preferred_element_type=jnp.float32) + bc2c_ref[...]


def _s4(xt, voxg, w0t, b0c, w1t, b1c, wct, bcc, wc2t, bc2c):
    nblk = (N + BN4 - 1) // BN4
    return pl.pallas_call(
        _s4_body,
        grid=(nblk,),
        in_specs=[
            pl.BlockSpec((D_IN, BN4), lambda i: (0, i)),
            pl.BlockSpec((BN4, 128), lambda i: (i, 0)),
            pl.BlockSpec((H, D_IN), lambda i: (0, 0)),
            pl.BlockSpec((H, 1), lambda i: (0, 0)),
            pl.BlockSpec((CS, H), lambda i: (0, 0)),
            pl.BlockSpec((CS, 1), lambda i: (0, 0)),
            pl.BlockSpec((NCLS, CS), lambda i: (0, 0)),
            pl.BlockSpec((NCLS, 1), lambda i: (0, 0)),
            pl.BlockSpec((NCLS, CS), lambda i: (0, 0)),
            pl.BlockSpec((NCLS, 1), lambda i: (0, 0)),
        ],
        out_specs=[
            pl.BlockSpec((CS, BN4), lambda i: (0, i)),
            pl.BlockSpec((NCLS, BN4), lambda i: (0, i)),
            pl.BlockSpec((NCLS, BN4), lambda i: (0, i)),
        ],
        out_shape=[
            jax.ShapeDtypeStruct((CS, N), jnp.float32),
            jax.ShapeDtypeStruct((NCLS, N), jnp.float32),
            jax.ShapeDtypeStruct((NCLS, N), jnp.float32),
        ],
    )(xt, voxg, w0t, b0c, w1t, b1c, wct, bcc, wc2t, bc2c)


@jax.jit
def kernel(pt_feats, voxel_idx, W0, b0, Wv, bv, W1, b1, Wc, bc, Wc2, bc2):
    idx = voxel_idx.astype(jnp.int32)
    idx_pad = jnp.concatenate(
        [idx, jnp.broadcast_to(idx[-1:], (N_PAD - N,))])
    idx2d = idx_pad.reshape(N_PAD // 128, 128)

    b0r = b0.reshape(1, H)
    bvr = bv.reshape(1, H)

    xt = jnp.pad(pt_feats.T, ((0, 0), (0, N_PAD - N)))
    cntp = _scnt(idx2d)
    h3 = _s0(xt, W0, b0r)
    vsum = _s1(h3, idx2d)
    voxp = _s2(vsum, cntp, Wv, bvr, W1)
    voxg = _s3(voxp, idx2d)
    ft, l1t, l2t = _s4(xt, voxg, W0.T, b0.reshape(H, 1),
                       W1.T, b1.reshape(CS, 1),
                       Wc.T, bc.reshape(NCLS, 1),
                       Wc2.T, bc2.reshape(NCLS, 1))
    return ft.T, l1t.T, l2t.T


# speedup vs baseline: 3.4238x; 1.0853x over previous
import jax
import jax.numpy as jnp
from jax import lax
from jax.experimental import pallas as pl
from jax.experimental.pallas import tpu as pltpu
from jax.experimental.pallas import tpu_sc as plsc

N = 100000
V = 20000
D_IN = 4
H = 256
CS = 96
NCLS = 19

N_PAD = 102400
V_PAD = 20480
BN0 = 4096
BN4 = 4096
BV = 1024

NC, NS = 2, 16
T1 = N_PAD // NS
C1 = T1 // 128
T3 = N_PAD // (NC * NS)
C3 = T3 // 128
VROWS = V_PAD // NS

_mesh = plsc.VectorSubcoreMesh(core_axis_name="c", subcore_axis_name="s")
_sc_params = pltpu.CompilerParams(use_tc_tiling_on_sc=False,
                                  needs_layout_passes=False)


def _s0_body(xt_ref, w0_ref, b0_ref, h_ref):
    i = pl.program_id(0)
    h = lax.dot_general(xt_ref[...], w0_ref[...],
                        (((0,), (0,)), ((), ())),
                        preferred_element_type=jnp.float32)
    h = jnp.maximum(h + b0_ref[...], 0.0)
    rows = i * BN0 + lax.broadcasted_iota(jnp.int32, (BN0, 1), 0)
    h = jnp.where(rows < N, h, 0.0)
    h_ref[0] = h[:, :128]
    h_ref[1] = h[:, 128:]


def _s0(xt, w0, b0r):
    return pl.pallas_call(
        _s0_body,
        grid=(N_PAD // BN0,),
        in_specs=[
            pl.BlockSpec((D_IN, BN0), lambda i: (0, i)),
            pl.BlockSpec((D_IN, H), lambda i: (0, 0)),
            pl.BlockSpec((1, H), lambda i: (0, 0)),
        ],
        out_specs=pl.BlockSpec((2, BN0, 128), lambda i: (0, i, 0)),
        out_shape=jax.ShapeDtypeStruct((2, N_PAD, 128), jnp.float32),
    )(xt, w0, b0r)


def _scnt_body(idx_hbm, cntp_hbm, idxbuf, cntbuf):
    c = lax.axis_index("c")
    s = lax.axis_index("s")
    wid = s * NC + c
    zero16 = jnp.zeros((16,), jnp.float32)
    ones = jnp.ones((16,), jnp.float32)
    iota = lax.iota(jnp.int32, 16)

    def _zcnt(i, _):
        cntbuf[pl.ds(i * 16, 16)] = zero16
        return 0
    lax.fori_loop(0, V_PAD // 16, _zcnt, 0)
    pltpu.sync_copy(idx_hbm.at[pl.ds(wid * C3, C3)], idxbuf)

    def _cchunk(j, _):
        def _cvec(k, _):
            gbase = wid * T3 + j * 128 + k * 16
            iv = idxbuf[j, pl.ds(k * 16, 16)]
            mask = (gbase + iota) < N
            plsc.addupdate_scatter(cntbuf, [iv], ones, mask=mask)
            return 0
        lax.fori_loop(0, 8, _cvec, 0)
        return 0
    lax.fori_loop(0, C3, _cchunk, 0)
    pltpu.sync_copy(cntbuf, cntp_hbm.at[wid])


def _scnt(idx2d):
    return pl.kernel(
        _scnt_body,
        out_type=jax.ShapeDtypeStruct((NC * NS, V_PAD), jnp.float32),
        mesh=_mesh,
        scratch_types=[
            pltpu.VMEM((C3, 128), jnp.int32),
            pltpu.VMEM((V_PAD,), jnp.float32),
        ],
        compiler_params=_sc_params,
    )(idx2d)


def _s1_body(h_hbm, idx_hbm, vsum_hbm,
             acc, hbuf0, hbuf1, idxbuf, zbuf, rsem0, rsem1):
    c = lax.axis_index("c")
    s = lax.axis_index("s")
    zero16 = jnp.zeros((16,), jnp.float32)
    bufs = (hbuf0, hbuf1)
    sems = (rsem0, rsem1)

    def _zrow(i, _):
        for k in range(4):
            zbuf[i, pl.ds(k * 16, 16)] = zero16
        return 0
    lax.fori_loop(0, 64, _zrow, 0)

    pltpu.sync_copy(idx_hbm.at[pl.ds(s * C1, C1)], idxbuf)

    for p in range(2):
        colbase = p * 64

        def _hsrc(j):
            return h_hbm.at[c, pl.ds(s * T1 + j * 128, 128),
                            pl.ds(colbase, 64)]

        def _zacc(i, _):
            pltpu.sync_copy(zbuf, acc.at[pl.ds(s * VROWS + i * 64, 64), :])
            return 0
        lax.fori_loop(0, VROWS // 64, _zacc, 0)
        plsc.subcore_barrier()

        pltpu.async_copy(_hsrc(0), hbuf0, rsem0)
        pltpu.async_copy(_hsrc(1), hbuf1, rsem1)

        def _pipe(j2, _):
            for b in range(2):
                j = 2 * j2 + b
                pltpu.make_async_copy(_hsrc(j), bufs[b], sems[b]).wait()
                pltpu.sync_copy(bufs[b], acc.at[idxbuf.at[j]], add=True)

                @pl.when(j + 2 < C1)
                def _():
                    pltpu.async_copy(_hsrc(j + 2), bufs[b], sems[b])
            return 0
        lax.fori_loop(0, C1 // 2, _pipe, 0)
        plsc.subcore_barrier()

        pltpu.sync_copy(
            acc.at[pl.ds(s * VROWS, VROWS), :],
            vsum_hbm.at[c, pl.ds(s * VROWS, VROWS), pl.ds(colbase, 64)])


def _s1(h3, idx2d):
    return pl.kernel(
        _s1_body,
        out_type=jax.ShapeDtypeStruct((2, V_PAD, 128), jnp.float32),
        mesh=_mesh,
        scratch_types=[
            pltpu.VMEM_SHARED((V_PAD, 64), jnp.float32),
            pltpu.VMEM((128, 64), jnp.float32),
            pltpu.VMEM((128, 64), jnp.float32),
            pltpu.VMEM((C1, 128), jnp.int32),
            pltpu.VMEM((64, 64), jnp.float32),
            pltpu.SemaphoreType.DMA,
            pltpu.SemaphoreType.DMA,
        ],
        compiler_params=_sc_params,
    )(h3, idx2d)


def _s2_body(vsum_ref, cntp_ref, wv_ref, bv_ref, w1_ref, voxp_ref):
    i = pl.program_id(0)
    cnt = jnp.sum(cntp_ref[:, pl.ds(i * BV, BV)], axis=0)[:, None]
    vs = jnp.concatenate([vsum_ref[0], vsum_ref[1]], axis=1)
    mean = vs / jnp.maximum(cnt, 1.0)
    vox = jnp.dot(mean, wv_ref[...], preferred_element_type=jnp.float32)
    vox = jnp.maximum(vox + bv_ref[...], 0.0)
    voxp_ref[...] = jnp.dot(vox, w1_ref[...],
                            preferred_element_type=jnp.float32)


def _s2(vsum, cntp, wv, bvr, w1):
    return pl.pallas_call(
        _s2_body,
        grid=(V_PAD // BV,),
        in_specs=[
            pl.BlockSpec((2, BV, 128), lambda i: (0, i, 0)),
            pl.BlockSpec((NC * NS, V_PAD), lambda i: (0, 0)),
            pl.BlockSpec((H, H), lambda i: (0, 0)),
            pl.BlockSpec((1, H), lambda i: (0, 0)),
            pl.BlockSpec((H, CS), lambda i: (0, 0)),
        ],
        out_specs=pl.BlockSpec((BV, CS), lambda i: (i, 0)),
        out_shape=jax.ShapeDtypeStruct((V_PAD, CS), jnp.float32),
    )(vsum, cntp, wv, bvr, w1)


_NB3 = 4


def _s3_body(voxp_hbm, idx_hbm, voxg_hbm, idxbuf,
             gbuf0, gbuf1, gbuf2, gbuf3,
             gsem0, gsem1, gsem2, gsem3,
             wsem0, wsem1, wsem2, wsem3):
    c = lax.axis_index("c")
    s = lax.axis_index("s")
    wid = s * NC + c
    bufs = (gbuf0, gbuf1, gbuf2, gbuf3)
    gsems = (gsem0, gsem1, gsem2, gsem3)
    wsems = (wsem0, wsem1, wsem2, wsem3)
    pltpu.sync_copy(idx_hbm.at[pl.ds(wid * C3, C3)], idxbuf)

    def _dst(j):
        return voxg_hbm.at[pl.ds(wid * T3 + j * 128, 128), pl.ds(0, CS)]

    for b in range(_NB3):
        pltpu.async_copy(voxp_hbm.at[idxbuf.at[b]], bufs[b], gsems[b])

    def _pipe(j2, _):
        for b in range(_NB3):
            j = _NB3 * j2 + b

            @pl.when(j < C3)
            def _():
                pltpu.make_async_copy(
                    voxp_hbm.at[idxbuf.at[j]], bufs[b], gsems[b]).wait()
                pltpu.async_copy(bufs[b], _dst(j), wsems[b])

                @pl.when(j + _NB3 < C3)
                def _():
                    pltpu.make_async_copy(bufs[b], _dst(j), wsems[b]).wait()
                    pltpu.async_copy(
                        voxp_hbm.at[idxbuf.at[j + _NB3]], bufs[b], gsems[b])
        return 0
    lax.fori_loop(0, (C3 + _NB3 - 1) // _NB3, _pipe, 0)

    for b in range(_NB3):
        j_last = ((C3 - 1 - b) // _NB3) * _NB3 + b
        pltpu.make_async_copy(bufs[b], _dst(j_last), wsems[b]).wait()


def _s3(voxp, idx2d):
    return pl.kernel(
        _s3_body,
        out_type=jax.ShapeDtypeStruct((N_PAD, 128), jnp.float32),
        mesh=_mesh,
        scratch_types=[
            pltpu.VMEM((C3, 128), jnp.int32),
            pltpu.VMEM((128, CS), jnp.float32),
            pltpu.VMEM((128, CS), jnp.float32),
            pltpu.VMEM((128, CS), jnp.float32),
            pltpu.VMEM((128, CS), jnp.float32),
            pltpu.SemaphoreType.DMA,
            pltpu.SemaphoreType.DMA,
            pltpu.SemaphoreType.DMA,
            pltpu.SemaphoreType.DMA,
            pltpu.SemaphoreType.DMA,
            pltpu.SemaphoreType.DMA,
            pltpu.SemaphoreType.DMA,
            pltpu.SemaphoreType.DMA,
        ],
        compiler_params=_sc_params,
    )(voxp, idx2d)


def _s4_body(xt_ref, voxg_ref, w0t_ref, b0c_ref, w1t_ref, b1c_ref,
             wct_ref, bcc_ref, wc2t_ref, bc2c_ref,
             feats_ref, l1_ref, l2_ref):
    ht = b0c_ref[...]
    for k in range(D_IN):
        ht = ht + w0t_ref[:, k:k + 1] * xt_ref[k:k + 1, :]
    ht = jnp.maximum(ht, 0.0)
    vgt = voxg_ref[...].T
    t = jnp.dot(w1t_ref[...], ht, preferred_element_type=jnp.float32)
    feats = jnp.maximum(t + vgt[:CS, :] + b1c_ref[...], 0.0)
    feats_ref[...] = feats
    l1_ref[...] = jnp.dot(wct_ref[...], feats,
                          preferred_element_type=jnp.float32) + bcc_ref[...]
    l2_ref[...] = jnp.dot(wc2t_ref[...], feats,
                          preferred_element_type=jnp.float32) + bc2c_ref[...]


def _s4(xt, voxg, w0t, b0c, w1t, b1c, wct, bcc, wc2t, bc2c):
    nblk = (N + BN4 - 1) // BN4
    return pl.pallas_call(
        _s4_body,
        grid=(nblk,),
        in_specs=[
            pl.BlockSpec((D_IN, BN4), lambda i: (0, i)),
            pl.BlockSpec((BN4, 128), lambda i: (i, 0)),
            pl.BlockSpec((H, D_IN), lambda i: (0, 0)),
            pl.BlockSpec((H, 1), lambda i: (0, 0)),
            pl.BlockSpec((CS, H), lambda i: (0, 0)),
            pl.BlockSpec((CS, 1), lambda i: (0, 0)),
            pl.BlockSpec((NCLS, CS), lambda i: (0, 0)),
            pl.BlockSpec((NCLS, 1), lambda i: (0, 0)),
            pl.BlockSpec((NCLS, CS), lambda i: (0, 0)),
            pl.BlockSpec((NCLS, 1), lambda i: (0, 0)),
        ],
        out_specs=[
            pl.BlockSpec((CS, BN4), lambda i: (0, i)),
            pl.BlockSpec((NCLS, BN4), lambda i: (0, i)),
            pl.BlockSpec((NCLS, BN4), lambda i: (0, i)),
        ],
        out_shape=[
            jax.ShapeDtypeStruct((CS, N), jnp.float32),
            jax.ShapeDtypeStruct((NCLS, N), jnp.float32),
            jax.ShapeDtypeStruct((NCLS, N), jnp.float32),
        ],
    )(xt, voxg, w0t, b0c, w1t, b1c, wct, bcc, wc2t, bc2c)


@jax.jit
def kernel(pt_feats, voxel_idx, W0, b0, Wv, bv, W1, b1, Wc, bc, Wc2, bc2):
    idx = voxel_idx.astype(jnp.int32)
    idx_pad = jnp.concatenate(
        [idx, jnp.broadcast_to(idx[-1:], (N_PAD - N,))])
    idx2d = idx_pad.reshape(N_PAD // 128, 128)

    b0r = b0.reshape(1, H)
    bvr = bv.reshape(1, H)

    xt = jnp.pad(pt_feats.T, ((0, 0), (0, N_PAD - N)))
    cntp = _scnt(idx2d)
    h3 = _s0(xt, W0, b0r)
    vsum = _s1(h3, idx2d)
    voxp = _s2(vsum, cntp, Wv, bvr, W1)
    voxg = _s3(voxp, idx2d)
    ft, l1t, l2t = _s4(xt, voxg, W0.T, b0.reshape(H, 1),
                       W1.T, b1.reshape(CS, 1),
                       Wc.T, bc.reshape(NCLS, 1),
                       Wc2.T, bc2.reshape(NCLS, 1))
    return ft.T, l1t.T, l2t.T
